# baseline (speedup 1.0000x reference)
import functools

import jax
import jax.numpy as jnp
import numpy as np
from jax import lax
from jax.experimental import pallas as pl
from jax.experimental.pallas import tpu as pltpu
from jax.experimental.pallas import tpu_sc as plsc

N, D = 10000, 128
NC, NS = 2, 16
NW = NC * NS
CHUNK = 128
CPT0 = 80
CPT1 = 80
SS = 40
TOT_CHUNKS = NS * (CPT0 + CPT1)
E_PAD = TOT_CHUNKS * CHUNK
N_PAD = 10112
ROWS_PER_SUB = N_PAD // NS
DUMP = N


def _sc_agg_body(edge_hbm, x_hbm, out_hbm, rowv, colv, gb0, gb1, agg,
                 sem0, sem1, ssem0, ssem1):
    c = lax.axis_index("c")
    s = lax.axis_index("s")
    base = jnp.where(c == 0, s * CPT0, NS * CPT0 + s * CPT1)
    nstages = jnp.where(c == 0, CPT0 // SS, CPT1 // SS)

    @pl.loop(0, CHUNK)
    def _zero(i):
        for j in range(CHUNK // 16):
            gb0[i, pl.ds(j * 16, 16)] = jnp.zeros((16,), jnp.float32)

    nfull, tail = divmod(ROWS_PER_SUB, CHUNK)
    for r in range(nfull):
        pltpu.sync_copy(gb0, agg.at[pl.ds(s * ROWS_PER_SUB + r * CHUNK, CHUNK)])
    if tail:
        pltpu.sync_copy(gb0.at[pl.ds(0, tail)],
                        agg.at[pl.ds(s * ROWS_PER_SUB + nfull * CHUNK, tail)])
    plsc.subcore_barrier()

    @pl.loop(0, nstages)
    def _stage(st):
        pltpu.sync_copy(edge_hbm.at[0, pl.ds(base + st * SS, SS)], rowv)
        pltpu.sync_copy(edge_hbm.at[1, pl.ds(base + st * SS, SS)], colv)

        @pl.loop(0, SS)
        def _mask(i):
            for j in range(CHUNK // 16):
                sl = pl.ds(j * 16, 16)
                r = rowv[i, sl]
                cc = colv[i, sl]
                dumpv = DUMP + (i % 6) * 16 + lax.iota(jnp.int32, 16)
                rowv[i, sl] = jnp.where(r == cc, dumpv, r)

        pltpu.async_copy(x_hbm.at[colv.at[0]], gb0, sem0)
        pltpu.async_copy(x_hbm.at[colv.at[1]], gb1, sem1)

        @pl.loop(0, SS, step=2)
        def _main(j):
            pltpu.make_async_copy(x_hbm.at[pl.ds(0, CHUNK)], gb0, sem0).wait()
            pltpu.async_copy(gb0, agg.at[rowv.at[j]], ssem0, add=True)
            pltpu.make_async_copy(x_hbm.at[pl.ds(0, CHUNK)], gb1, sem1).wait()
            pltpu.async_copy(gb1, agg.at[rowv.at[j + 1]], ssem1, add=True)

            @pl.when(j + 2 < SS)
            def _():
                pltpu.make_async_copy(gb0, agg.at[pl.ds(0, CHUNK)], ssem0).wait()
                pltpu.async_copy(x_hbm.at[colv.at[j + 2]], gb0, sem0)

            @pl.when(j + 3 < SS)
            def _():
                pltpu.make_async_copy(gb1, agg.at[pl.ds(0, CHUNK)], ssem1).wait()
                pltpu.async_copy(x_hbm.at[colv.at[j + 3]], gb1, sem1)

        pltpu.make_async_copy(gb0, agg.at[pl.ds(0, CHUNK)], ssem0).wait()
        pltpu.make_async_copy(gb1, agg.at[pl.ds(0, CHUNK)], ssem1).wait()

    plsc.subcore_barrier()
    pltpu.sync_copy(agg.at[pl.ds(s * ROWS_PER_SUB, ROWS_PER_SUB)],
                    out_hbm.at[c, pl.ds(s * ROWS_PER_SUB, ROWS_PER_SUB)])


_sc_aggregate = pl.kernel(
    _sc_agg_body,
    out_type=jax.ShapeDtypeStruct((NC, N_PAD, D), jnp.float32),
    mesh=plsc.VectorSubcoreMesh(core_axis_name="c", subcore_axis_name="s"),
    scratch_types=[
        pltpu.VMEM((SS, CHUNK), jnp.int32),
        pltpu.VMEM((SS, CHUNK), jnp.int32),
        pltpu.VMEM((CHUNK, D), jnp.float32),
        pltpu.VMEM((CHUNK, D), jnp.float32),
        pltpu.VMEM_SHARED((N_PAD, D), jnp.float32),
        pltpu.SemaphoreType.DMA,
        pltpu.SemaphoreType.DMA,
        pltpu.SemaphoreType.DMA,
        pltpu.SemaphoreType.DMA,
    ],
)


def _tc_mlp_body(x_ref, agg_ref, w1_ref, b1_ref, w2_ref, b2_ref, g_ref, be_ref,
                 out_ref):
    h = x_ref[...] + agg_ref[0, :N, :] + agg_ref[1, :N, :]
    h = jax.lax.dot_general(h, w1_ref[...], (((1,), (0,)), ((), ())),
                            preferred_element_type=jnp.float32)
    h = jnp.maximum(h + b1_ref[...], 0.0)
    h = jax.lax.dot_general(h, w2_ref[...], (((1,), (0,)), ((), ())),
                            preferred_element_type=jnp.float32)
    h = jnp.maximum(h + b2_ref[...], 0.0)
    m = jnp.mean(h, axis=0, keepdims=True)
    v = jnp.mean(jnp.square(h - m), axis=0, keepdims=True)
    out_ref[...] = g_ref[...] * (h - m) * jax.lax.rsqrt(v + 1e-5) + be_ref[...]


_tc_mlp = pl.pallas_call(
    _tc_mlp_body,
    out_shape=jax.ShapeDtypeStruct((N, D), jnp.float32),
)


@jax.jit
def kernel(x, edge_index, W1, b1, W2, b2, gamma, beta):
    e = edge_index.shape[1]
    pad = E_PAD - e
    pad_blk = jnp.asarray(np.stack([
        DUMP + (np.arange(pad) % (N_PAD - N)),
        np.arange(pad) % N,
    ]).reshape(2, pad // CHUNK, CHUNK), jnp.int32)
    edge_p = jnp.concatenate(
        [edge_index.reshape(2, e // CHUNK, CHUNK), pad_blk], axis=1)
    agg = _sc_aggregate(edge_p, x)
    return _tc_mlp(x, agg, W1, b1.reshape(1, D), W2, b2.reshape(1, D),
                   gamma.reshape(1, D), beta.reshape(1, D))


# speedup vs baseline: 1.0426x; 1.0426x over previous
import functools

import jax
import jax.numpy as jnp
import numpy as np
from jax import lax
from jax.experimental import pallas as pl
from jax.experimental.pallas import tpu as pltpu
from jax.experimental.pallas import tpu_sc as plsc

N, D = 10000, 128
NC, NS = 2, 16
NW = NC * NS
CHUNK = 128
CPT0 = 80
CPT1 = 80
SS = 40
TOT_CHUNKS = NS * (CPT0 + CPT1)
E_PAD = TOT_CHUNKS * CHUNK
N_PAD = 10112
ROWS_PER_SUB = N_PAD // NS
DUMP = N


def _sc_agg_body(edge_hbm, x_hbm, out_hbm, rowv, colv, gb0, gb1, agg,
                 sem0, sem1):
    c = lax.axis_index("c")
    s = lax.axis_index("s")
    base = jnp.where(c == 0, s * CPT0, NS * CPT0 + s * CPT1)
    nstages = jnp.where(c == 0, CPT0 // SS, CPT1 // SS)

    @pl.loop(0, CHUNK)
    def _zero(i):
        for j in range(CHUNK // 16):
            gb0[i, pl.ds(j * 16, 16)] = jnp.zeros((16,), jnp.float32)

    nfull, tail = divmod(ROWS_PER_SUB, CHUNK)
    for r in range(nfull):
        pltpu.sync_copy(gb0, agg.at[pl.ds(s * ROWS_PER_SUB + r * CHUNK, CHUNK)])
    if tail:
        pltpu.sync_copy(gb0.at[pl.ds(0, tail)],
                        agg.at[pl.ds(s * ROWS_PER_SUB + nfull * CHUNK, tail)])
    plsc.subcore_barrier()

    @pl.loop(0, nstages)
    def _stage(st):
        pltpu.sync_copy(edge_hbm.at[0, pl.ds(base + st * SS, SS)], rowv)
        pltpu.sync_copy(edge_hbm.at[1, pl.ds(base + st * SS, SS)], colv)

        @pl.loop(0, SS)
        def _mask(i):
            for j in range(CHUNK // 16):
                sl = pl.ds(j * 16, 16)
                r = rowv[i, sl]
                cc = colv[i, sl]
                dumpv = DUMP + (i % 6) * 16 + lax.iota(jnp.int32, 16)
                rowv[i, sl] = jnp.where(r == cc, dumpv, r)

        pltpu.async_copy(x_hbm.at[colv.at[0]], gb0, sem0)
        pltpu.async_copy(x_hbm.at[colv.at[1]], gb1, sem1)

        @pl.loop(0, SS, step=2)
        def _main(j):
            pltpu.make_async_copy(x_hbm.at[pl.ds(0, CHUNK)], gb0, sem0).wait()
            pltpu.sync_copy(gb0, agg.at[rowv.at[j]], add=True)

            @pl.when(j + 2 < SS)
            def _():
                pltpu.async_copy(x_hbm.at[colv.at[j + 2]], gb0, sem0)

            pltpu.make_async_copy(x_hbm.at[pl.ds(0, CHUNK)], gb1, sem1).wait()
            pltpu.sync_copy(gb1, agg.at[rowv.at[j + 1]], add=True)

            @pl.when(j + 3 < SS)
            def _():
                pltpu.async_copy(x_hbm.at[colv.at[j + 3]], gb1, sem1)

    plsc.subcore_barrier()
    pltpu.sync_copy(agg.at[pl.ds(s * ROWS_PER_SUB, ROWS_PER_SUB)],
                    out_hbm.at[c, pl.ds(s * ROWS_PER_SUB, ROWS_PER_SUB)])


_sc_aggregate = pl.kernel(
    _sc_agg_body,
    out_type=jax.ShapeDtypeStruct((NC, N_PAD, D), jnp.float32),
    mesh=plsc.VectorSubcoreMesh(core_axis_name="c", subcore_axis_name="s"),
    scratch_types=[
        pltpu.VMEM((SS, CHUNK), jnp.int32),
        pltpu.VMEM((SS, CHUNK), jnp.int32),
        pltpu.VMEM((CHUNK, D), jnp.float32),
        pltpu.VMEM((CHUNK, D), jnp.float32),
        pltpu.VMEM_SHARED((N_PAD, D), jnp.float32),
        pltpu.SemaphoreType.DMA,
        pltpu.SemaphoreType.DMA,
    ],
)


def _tc_mlp_body(x_ref, a0_ref, a1_ref, w1_ref, b1_ref, w2_ref, b2_ref,
                 g_ref, be_ref, h_in_ref, h_ref, out_ref, acc_ref):
    p = pl.program_id(0)
    b = pl.program_id(1)

    @pl.when(p == 0)
    def _():
        h = x_ref[...] + a0_ref[0] + a1_ref[0]
        h = jax.lax.dot_general(h, w1_ref[...], (((1,), (0,)), ((), ())),
                                preferred_element_type=jnp.float32)
        h = jnp.maximum(h + b1_ref[...], 0.0)
        h = jax.lax.dot_general(h, w2_ref[...], (((1,), (0,)), ((), ())),
                                preferred_element_type=jnp.float32)
        h = jnp.maximum(h + b2_ref[...], 0.0)
        h_ref[...] = h

        @pl.when(b == 0)
        def _():
            acc_ref[...] = jnp.zeros_like(acc_ref)

        acc_ref[0] += jnp.sum(h, axis=0)

    @pl.when(p == 1)
    def _():
        m = acc_ref[0] * (1.0 / N)
        acc_ref[1] += jnp.sum(jnp.square(h_in_ref[...] - m), axis=0)

    @pl.when(p == 2)
    def _():
        m = acc_ref[0] * (1.0 / N)
        v = acc_ref[1] * (1.0 / N)
        scale = g_ref[0] * jax.lax.rsqrt(v + 1e-5)
        out_ref[...] = h_in_ref[...] * scale + (be_ref[0] - m * scale)


NBLK = 10
BLK = N // NBLK

_tc_mlp = pl.pallas_call(
    _tc_mlp_body,
    grid=(3, NBLK),
    in_specs=[
        pl.BlockSpec((BLK, D), lambda p, b: (b, 0)),
        pl.BlockSpec((1, BLK, D), lambda p, b: (0, b, 0)),
        pl.BlockSpec((1, BLK, D), lambda p, b: (1, b, 0)),
        pl.BlockSpec((D, D), lambda p, b: (0, 0)),
        pl.BlockSpec((1, D), lambda p, b: (0, 0)),
        pl.BlockSpec((D, D), lambda p, b: (0, 0)),
        pl.BlockSpec((1, D), lambda p, b: (0, 0)),
        pl.BlockSpec((1, D), lambda p, b: (0, 0)),
        pl.BlockSpec((1, D), lambda p, b: (0, 0)),
        pl.BlockSpec((BLK, D), lambda p, b: (b, 0)),
    ],
    out_specs=[
        pl.BlockSpec((BLK, D), lambda p, b: (b, 0)),
        pl.BlockSpec((BLK, D), lambda p, b: (b, 0)),
    ],
    out_shape=[
        jax.ShapeDtypeStruct((N, D), jnp.float32),
        jax.ShapeDtypeStruct((N, D), jnp.float32),
    ],
    scratch_shapes=[pltpu.VMEM((2, D), jnp.float32)],
    input_output_aliases={9: 0},
)


@jax.jit
def kernel(x, edge_index, W1, b1, W2, b2, gamma, beta):
    e = edge_index.shape[1]
    pad = E_PAD - e
    pad_blk = jnp.asarray(np.stack([
        DUMP + (np.arange(pad) % (N_PAD - N)),
        np.arange(pad) % N,
    ]).reshape(2, pad // CHUNK, CHUNK), jnp.int32)
    edge_p = jnp.concatenate(
        [edge_index.reshape(2, e // CHUNK, CHUNK), pad_blk], axis=1)
    agg = _sc_aggregate(edge_p, x)
    h0 = jnp.zeros((N, D), jnp.float32)
    _, out = _tc_mlp(x, agg, agg, W1, b1.reshape(1, D), W2,
                     b2.reshape(1, D), gamma.reshape(1, D),
                     beta.reshape(1, D), h0)
    return out


# speedup vs baseline: 1.2300x; 1.1797x over previous
import functools

import jax
import jax.numpy as jnp
import numpy as np
from jax import lax
from jax.experimental import pallas as pl
from jax.experimental.pallas import tpu as pltpu
from jax.experimental.pallas import tpu_sc as plsc

N, D = 10000, 128
NC, NS = 2, 16
NW = NC * NS
CHUNK = 128
CPT0 = 80
CPT1 = 80
SS = 40
TOT_CHUNKS = NS * (CPT0 + CPT1)
E_PAD = TOT_CHUNKS * CHUNK
N_PAD = 10112
ROWS_PER_SUB = N_PAD // NS
DUMP = N


def _sc_agg_body(edge_hbm, x_hbm, out_hbm, rowv, colv, gb0, gb1, agg,
                 sem0, sem1, sem0b, sem1b):
    c = lax.axis_index("c")
    s = lax.axis_index("s")
    base = jnp.where(c == 0, s * CPT0, NS * CPT0 + s * CPT1)
    nstages = jnp.where(c == 0, CPT0 // SS, CPT1 // SS)

    @pl.loop(0, CHUNK)
    def _zero(i):
        for j in range(CHUNK // 16):
            gb0[i, pl.ds(j * 16, 16)] = jnp.zeros((16,), jnp.float32)

    nfull, tail = divmod(ROWS_PER_SUB, CHUNK)
    for r in range(nfull):
        pltpu.sync_copy(gb0, agg.at[pl.ds(s * ROWS_PER_SUB + r * CHUNK, CHUNK)])
    if tail:
        pltpu.sync_copy(gb0.at[pl.ds(0, tail)],
                        agg.at[pl.ds(s * ROWS_PER_SUB + nfull * CHUNK, tail)])
    plsc.subcore_barrier()

    @pl.loop(0, nstages)
    def _stage(st):
        pltpu.sync_copy(edge_hbm.at[0, pl.ds(base + st * SS, SS)], rowv)
        pltpu.sync_copy(edge_hbm.at[1, pl.ds(base + st * SS, SS)], colv)

        @pl.loop(0, SS)
        def _mask(i):
            for j in range(CHUNK // 16):
                sl = pl.ds(j * 16, 16)
                r = rowv[i, sl]
                cc = colv[i, sl]
                dumpv = DUMP + (i % 6) * 16 + lax.iota(jnp.int32, 16)
                rowv[i, sl] = jnp.where(r == cc, dumpv, r)

        H = CHUNK // 2

        def _gather2(j, gb, sa, sb):
            pltpu.async_copy(x_hbm.at[colv.at[j, pl.ds(0, H)]],
                             gb.at[pl.ds(0, H)], sa)
            pltpu.async_copy(x_hbm.at[colv.at[j, pl.ds(H, H)]],
                             gb.at[pl.ds(H, H)], sb)

        def _wait2(gb, sa, sb):
            pltpu.make_async_copy(x_hbm.at[pl.ds(0, H)], gb.at[pl.ds(0, H)],
                                  sa).wait()
            pltpu.make_async_copy(x_hbm.at[pl.ds(0, H)], gb.at[pl.ds(H, H)],
                                  sb).wait()

        _gather2(0, gb0, sem0, sem0b)
        _gather2(1, gb1, sem1, sem1b)

        @pl.loop(0, SS, step=2)
        def _main(j):
            _wait2(gb0, sem0, sem0b)
            pltpu.sync_copy(gb0, agg.at[rowv.at[j]], add=True)

            @pl.when(j + 2 < SS)
            def _():
                _gather2(j + 2, gb0, sem0, sem0b)

            _wait2(gb1, sem1, sem1b)
            pltpu.sync_copy(gb1, agg.at[rowv.at[j + 1]], add=True)

            @pl.when(j + 3 < SS)
            def _():
                _gather2(j + 3, gb1, sem1, sem1b)

    plsc.subcore_barrier()
    pltpu.sync_copy(agg.at[pl.ds(s * ROWS_PER_SUB, ROWS_PER_SUB)],
                    out_hbm.at[c, pl.ds(s * ROWS_PER_SUB, ROWS_PER_SUB)])


_sc_aggregate = pl.kernel(
    _sc_agg_body,
    out_type=jax.ShapeDtypeStruct((NC, N_PAD, D), jnp.float32),
    mesh=plsc.VectorSubcoreMesh(core_axis_name="c", subcore_axis_name="s"),
    scratch_types=[
        pltpu.VMEM((SS, CHUNK), jnp.int32),
        pltpu.VMEM((SS, CHUNK), jnp.int32),
        pltpu.VMEM((CHUNK, D), jnp.float32),
        pltpu.VMEM((CHUNK, D), jnp.float32),
        pltpu.VMEM_SHARED((N_PAD, D), jnp.float32),
        pltpu.SemaphoreType.DMA,
        pltpu.SemaphoreType.DMA,
        pltpu.SemaphoreType.DMA,
        pltpu.SemaphoreType.DMA,
    ],
)


def _tc_mlp_body(x_ref, agg_ref, w1_ref, b1_ref, w2_ref, b2_ref, g_ref, be_ref,
                 out_ref):
    h = x_ref[...] + agg_ref[0, :N, :] + agg_ref[1, :N, :]
    h = jax.lax.dot_general(h, w1_ref[...], (((1,), (0,)), ((), ())),
                            preferred_element_type=jnp.float32)
    h = jnp.maximum(h + b1_ref[...], 0.0)
    h = jax.lax.dot_general(h, w2_ref[...], (((1,), (0,)), ((), ())),
                            preferred_element_type=jnp.float32)
    h = jnp.maximum(h + b2_ref[...], 0.0)
    m = jnp.mean(h, axis=0, keepdims=True)
    v = jnp.mean(jnp.square(h - m), axis=0, keepdims=True)
    out_ref[...] = g_ref[...] * (h - m) * jax.lax.rsqrt(v + 1e-5) + be_ref[...]


_tc_mlp = pl.pallas_call(
    _tc_mlp_body,
    out_shape=jax.ShapeDtypeStruct((N, D), jnp.float32),
)


@jax.jit
def kernel(x, edge_index, W1, b1, W2, b2, gamma, beta):
    e = edge_index.shape[1]
    pad = E_PAD - e
    pad_blk = jnp.asarray(np.stack([
        DUMP + (np.arange(pad) % (N_PAD - N)),
        np.arange(pad) % N,
    ]).reshape(2, pad // CHUNK, CHUNK), jnp.int32)
    edge_p = jnp.concatenate(
        [edge_index.reshape(2, e // CHUNK, CHUNK), pad_blk], axis=1)
    agg = _sc_aggregate(edge_p, x)
    return _tc_mlp(x, agg, W1, b1.reshape(1, D), W2, b2.reshape(1, D),
                   gamma.reshape(1, D), beta.reshape(1, D))


# speedup vs baseline: 1.2516x; 1.0176x over previous
import functools

import jax
import jax.numpy as jnp
import numpy as np
from jax import lax
from jax.experimental import pallas as pl
from jax.experimental.pallas import tpu as pltpu
from jax.experimental.pallas import tpu_sc as plsc

N, D = 10000, 128
NC, NS = 2, 16
NW = NC * NS
CHUNK = 128
CPT0 = 80
CPT1 = 80
SS = 40
TOT_CHUNKS = NS * (CPT0 + CPT1)
E_PAD = TOT_CHUNKS * CHUNK
N_PAD = 10112
ROWS_PER_SUB = N_PAD // NS
DUMP = N


def _sc_agg_body(edge_hbm, x_hbm, out_hbm, rowv, colv, gb0, gb1, agg,
                 sem0, sem1):
    c = lax.axis_index("c")
    s = lax.axis_index("s")
    base = jnp.where(c == 0, s * CPT0, NS * CPT0 + s * CPT1)
    nstages = jnp.where(c == 0, CPT0 // SS, CPT1 // SS)

    @pl.loop(0, CHUNK)
    def _zero(i):
        for j in range(CHUNK // 16):
            gb0[i, pl.ds(j * 16, 16)] = jnp.zeros((16,), jnp.float32)

    nfull, tail = divmod(ROWS_PER_SUB, CHUNK)
    for r in range(nfull):
        pltpu.sync_copy(gb0, agg.at[pl.ds(s * ROWS_PER_SUB + r * CHUNK, CHUNK)])
    if tail:
        pltpu.sync_copy(gb0.at[pl.ds(0, tail)],
                        agg.at[pl.ds(s * ROWS_PER_SUB + nfull * CHUNK, tail)])
    plsc.subcore_barrier()

    @pl.loop(0, nstages)
    def _stage(st):
        pltpu.sync_copy(edge_hbm.at[0, pl.ds(base + st * SS, SS)], rowv)
        pltpu.sync_copy(edge_hbm.at[1, pl.ds(base + st * SS, SS)], colv)

        @pl.loop(0, SS)
        def _mask(i):
            for j in range(CHUNK // 16):
                sl = pl.ds(j * 16, 16)
                r = rowv[i, sl]
                cc = colv[i, sl]
                dumpv = DUMP + (i % 6) * 16 + lax.iota(jnp.int32, 16)
                rowv[i, sl] = jnp.where(r == cc, dumpv, r)

        pltpu.async_copy(x_hbm.at[colv.at[0]], gb0, sem0)
        pltpu.async_copy(x_hbm.at[colv.at[1]], gb1, sem1)

        @pl.loop(0, SS, step=2)
        def _main(j):
            pltpu.make_async_copy(x_hbm.at[pl.ds(0, CHUNK)], gb0, sem0).wait()
            pltpu.sync_copy(gb0, agg.at[rowv.at[j]], add=True)

            @pl.when(j + 2 < SS)
            def _():
                pltpu.async_copy(x_hbm.at[colv.at[j + 2]], gb0, sem0)

            pltpu.make_async_copy(x_hbm.at[pl.ds(0, CHUNK)], gb1, sem1).wait()
            pltpu.sync_copy(gb1, agg.at[rowv.at[j + 1]], add=True)

            @pl.when(j + 3 < SS)
            def _():
                pltpu.async_copy(x_hbm.at[colv.at[j + 3]], gb1, sem1)

    plsc.subcore_barrier()
    pltpu.sync_copy(agg.at[pl.ds(s * ROWS_PER_SUB, ROWS_PER_SUB)],
                    out_hbm.at[c, pl.ds(s * ROWS_PER_SUB, ROWS_PER_SUB)])


_sc_aggregate = pl.kernel(
    _sc_agg_body,
    out_type=jax.ShapeDtypeStruct((NC, N_PAD, D), jnp.float32),
    mesh=plsc.VectorSubcoreMesh(core_axis_name="c", subcore_axis_name="s"),
    scratch_types=[
        pltpu.VMEM((SS, CHUNK), jnp.int32),
        pltpu.VMEM((SS, CHUNK), jnp.int32),
        pltpu.VMEM((CHUNK, D), jnp.float32),
        pltpu.VMEM((CHUNK, D), jnp.float32),
        pltpu.VMEM_SHARED((N_PAD, D), jnp.float32),
        pltpu.SemaphoreType.DMA,
        pltpu.SemaphoreType.DMA,
    ],
)


def _tc_mlp_body(x_ref, agg_ref, w1_ref, b1_ref, w2_ref, b2_ref, g_ref, be_ref,
                 out_ref):
    h = x_ref[...] + agg_ref[0, :N, :] + agg_ref[1, :N, :]
    h = jax.lax.dot_general(h, w1_ref[...], (((1,), (0,)), ((), ())),
                            preferred_element_type=jnp.float32)
    h = jnp.maximum(h + b1_ref[...], 0.0)
    h = jax.lax.dot_general(h, w2_ref[...], (((1,), (0,)), ((), ())),
                            preferred_element_type=jnp.float32)
    h = jnp.maximum(h + b2_ref[...], 0.0)
    m = jnp.mean(h, axis=0, keepdims=True)
    v = jnp.mean(jnp.square(h - m), axis=0, keepdims=True)
    out_ref[...] = g_ref[...] * (h - m) * jax.lax.rsqrt(v + 1e-5) + be_ref[...]


_tc_mlp = pl.pallas_call(
    _tc_mlp_body,
    out_shape=jax.ShapeDtypeStruct((N, D), jnp.float32),
)


@jax.jit
def kernel(x, edge_index, W1, b1, W2, b2, gamma, beta):
    e = edge_index.shape[1]
    pad = E_PAD - e
    pad_blk = jnp.asarray(np.stack([
        DUMP + (np.arange(pad) % (N_PAD - N)),
        np.arange(pad) % N,
    ]).reshape(2, pad // CHUNK, CHUNK), jnp.int32)
    edge_p = jnp.concatenate(
        [edge_index.reshape(2, e // CHUNK, CHUNK), pad_blk], axis=1)
    agg = _sc_aggregate(edge_p, x)
    return _tc_mlp(x, agg, W1, b1.reshape(1, D), W2, b2.reshape(1, D),
                   gamma.reshape(1, D), beta.reshape(1, D))
